# baseline (speedup 1.0000x reference)
import functools

import jax
import jax.numpy as jnp
from jax.experimental import pallas as pl
from jax.experimental.pallas import tpu as pltpu
from jax.experimental.pallas import tpu_sc as plsc

N = 128
A = 256
C = 1000
BETA = 1.0
GWIN = 8


def _sc_gather_rows(kg_sigma, idx_row):
    half = N // 2

    @pl.kernel(
        out_type=jax.ShapeDtypeStruct((N, C), jnp.float32),
        mesh=plsc.ScalarSubcoreMesh(axis_name="core", num_cores=2),
        scratch_types=[pltpu.SMEM((N,), jnp.int32),
                       pltpu.SemaphoreType.DMA,
                       pltpu.SemaphoreType.DMA],
    )
    def _gather(kg_hbm, i_hbm, o_hbm, idx_sm, isem, sem):
        core = jax.lax.axis_index("core")
        pltpu.async_copy(i_hbm.at[0], idx_sm, isem).wait()

        @pl.loop(0, half)
        def _(i):
            k = core * half + i
            pltpu.async_copy(kg_hbm.at[idx_sm[k]], o_hbm.at[k], sem)

        @pl.loop(0, half)
        def _(i):
            k = core * half + i
            pltpu.make_async_copy(kg_hbm.at[idx_sm[k]], o_hbm.at[k],
                                  sem).wait()

    return _gather(kg_sigma, idx_row)


def _isda_body(labels_ref, tail_ref, wts_ref, x_ref, fc_ref, kgl_ref,
               loss_ref, y_ref):
    F = x_ref[...]
    W = fc_ref[...]
    labels = labels_ref[...]
    tail = tail_ref[...]
    wts = wts_ref[...]

    cls_iota = jax.lax.broadcasted_iota(jnp.int32, (N, C), 1)
    onehot = (cls_iota == labels).astype(jnp.float32)

    dot_t = functools.partial(
        jax.lax.dot_general,
        dimension_numbers=(((1,), (1,)), ((), ())),
        preferred_element_type=jnp.float32,
    )

    P = dot_t(onehot, onehot)
    cnt = jnp.sum(P, axis=1, keepdims=True)
    mean = jnp.dot(P, F, preferred_element_type=jnp.float32) / cnt
    H = (F - mean) ** 2 / cnt

    Bp = dot_t(kgl_ref[...], onehot)

    in_tail = jnp.max((labels == tail).astype(jnp.float32),
                      axis=1, keepdims=True)
    mixer = jnp.where(in_tail > 0, Bp, P)
    U = jnp.dot(mixer, H, preferred_element_type=jnp.float32)

    Wl = jnp.dot(onehot, W, preferred_element_type=jnp.float32)
    V = U * Wl
    s = jnp.sum(V * Wl, axis=1, keepdims=True)

    y = dot_t(F, W)
    Vw = dot_t(V, W)
    Uw2 = dot_t(U, W * W)
    Z = y + BETA * (0.5 * Uw2 - Vw + 0.5 * s)

    m = jnp.max(Z, axis=1, keepdims=True)
    lse = m + jnp.log(jnp.sum(jnp.exp(Z - m), axis=1, keepdims=True))
    z_lab = jnp.sum(Z * onehot, axis=1, keepdims=True)
    w_lab = jnp.sum(wts * onehot, axis=1, keepdims=True)
    nll = lse - z_lab
    loss_ref[...] = (jnp.sum(w_lab * nll, keepdims=True)
                     / jnp.sum(w_lab, keepdims=True))
    y_ref[...] = y


@jax.jit
def kernel(x, target_x, weights, kg_sigma, index_tail, fc_weight):
    labels = target_x.reshape(N, 1)
    tail = index_tail.reshape(1, -1)
    wts = weights.reshape(1, C)

    kgl = _sc_gather_rows(kg_sigma, target_x.reshape(1, N))

    loss, y = pl.pallas_call(
        _isda_body,
        out_shape=(
            jax.ShapeDtypeStruct((1, 1), jnp.float32),
            jax.ShapeDtypeStruct((N, C), jnp.float32),
        ),
    )(labels, tail, wts, x, fc_weight, kgl)
    return (loss[0, 0], y)


# speedup vs baseline: 4.4261x; 4.4261x over previous
import functools

import jax
import jax.numpy as jnp
from jax.experimental import pallas as pl

N = 128
A = 256
C = 1000
BETA = 1.0


def _isda_body(labels_ref, tail_ref, wts_ref, x_ref, fc_ref, kg_ref,
               loss_ref, y_ref):
    F = x_ref[...]
    W = fc_ref[...]
    labels = labels_ref[...]
    tail = tail_ref[...]
    wts = wts_ref[...]

    cls_iota = jax.lax.broadcasted_iota(jnp.int32, (N, C), 1)
    onehot = (cls_iota == labels).astype(jnp.float32)

    dot_t = functools.partial(
        jax.lax.dot_general,
        dimension_numbers=(((1,), (1,)), ((), ())),
        preferred_element_type=jnp.float32,
    )

    onehot_h = onehot.astype(jnp.bfloat16)
    P = dot_t(onehot_h, onehot_h)
    cnt = jnp.sum(P, axis=1, keepdims=True)
    mean = jnp.dot(P, F, preferred_element_type=jnp.float32) / cnt
    H = (F - mean) ** 2 / cnt

    kgl = jnp.dot(onehot, kg_ref[...],
                  preferred_element_type=jnp.float32)
    Bp = dot_t(kgl, onehot)

    in_tail = jnp.max((labels == tail).astype(jnp.float32),
                      axis=1, keepdims=True)
    mixer = jnp.where(in_tail > 0, Bp, P)
    U = jnp.dot(mixer, H, preferred_element_type=jnp.float32)

    Wl = jnp.dot(onehot, W, preferred_element_type=jnp.float32)
    V = U * Wl
    s = jnp.sum(V * Wl, axis=1, keepdims=True)

    y = dot_t(F, W)
    Vw = dot_t(V, W)
    Uw2 = dot_t(U, W * W)
    Z = y + BETA * (0.5 * Uw2 - Vw + 0.5 * s)

    m = jnp.max(Z, axis=1, keepdims=True)
    lse = m + jnp.log(jnp.sum(jnp.exp(Z - m), axis=1, keepdims=True))
    z_lab = jnp.sum(Z * onehot, axis=1, keepdims=True)
    w_lab = jnp.sum(wts * onehot, axis=1, keepdims=True)
    nll = lse - z_lab
    loss_ref[...] = (jnp.sum(w_lab * nll, keepdims=True)
                     / jnp.sum(w_lab, keepdims=True))
    y_ref[...] = y


@jax.jit
def kernel(x, target_x, weights, kg_sigma, index_tail, fc_weight):
    labels = target_x.reshape(N, 1)
    tail = index_tail.reshape(1, -1)
    wts = weights.reshape(1, C)
    loss, y = pl.pallas_call(
        _isda_body,
        out_shape=(
            jax.ShapeDtypeStruct((1, 1), jnp.float32),
            jax.ShapeDtypeStruct((N, C), jnp.float32),
        ),
    )(labels, tail, wts, x, fc_weight, kg_sigma)
    return (loss[0, 0], y)
